# baseline (speedup 1.0000x reference)
import functools

import jax
import jax.numpy as jnp
from jax.experimental import pallas as pl
from jax.experimental.pallas import tpu as pltpu

NUM_EXPERTS = 8
TOP_K = 2
RANK = 128
D_IN = 2048
D_OUT = 2048
BLK = 512


def _moe_body(x_ref, wg_ref, wn_ref, wd_ref, wu_ref, noise_ref,
              out_ref, rl_ref):
    x = x_ref[...]

    logits = jax.lax.dot_general(
        x, wg_ref[...], (((1,), (1,)), ((), ())),
        preferred_element_type=jnp.float32)
    nlogits = jax.lax.dot_general(
        x, wn_ref[...], (((1,), (1,)), ((), ())),
        preferred_element_type=jnp.float32)
    rl = logits + noise_ref[...] * jax.nn.softplus(nlogits)
    rl_ref[...] = rl

    p = jax.nn.softmax(rl, axis=-1)

    col = jax.lax.broadcasted_iota(jnp.int32, p.shape, 1)
    m1 = jnp.max(p, axis=-1, keepdims=True)
    a1 = jnp.min(jnp.where(p == m1, col, NUM_EXPERTS), axis=-1, keepdims=True)
    first = col == a1
    p_m = jnp.where(first, -jnp.inf, p)
    m2 = jnp.max(p_m, axis=-1, keepdims=True)
    a2 = jnp.min(jnp.where(p_m == m2, col, NUM_EXPERTS), axis=-1, keepdims=True)
    sel = first | (col == a2)
    w = jnp.where(sel, p, 0.0)
    w = w / jnp.sum(w, axis=-1, keepdims=True)

    xb = x.astype(jnp.bfloat16)
    acc = jnp.zeros((x.shape[0], D_OUT), jnp.float32)
    for e in range(NUM_EXPERTS):
        down = jax.lax.dot_general(
            xb, wd_ref[e], (((1,), (1,)), ((), ())),
            preferred_element_type=jnp.float32)
        up = jax.lax.dot_general(
            down.astype(jnp.bfloat16), wu_ref[e], (((1,), (1,)), ((), ())),
            preferred_element_type=jnp.float32)
        acc = acc + up * w[:, e:e + 1]
    out_ref[...] = acc


@functools.partial(jax.jit, static_argnames=("interpret",))
def kernel(hidden_states, Wg, Wn, W_down, W_up, interpret=False):
    b, s, d = hidden_states.shape
    T = b * s
    x = hidden_states.reshape(T, d)
    noise = jax.random.normal(jax.random.key(42), (T, NUM_EXPERTS),
                              jnp.float32)

    grid = (T // BLK,)
    out, rl = pl.pallas_call(
        _moe_body,
        grid=grid,
        in_specs=[
            pl.BlockSpec((BLK, D_IN), lambda i: (i, 0)),
            pl.BlockSpec((NUM_EXPERTS, D_IN), lambda i: (0, 0)),
            pl.BlockSpec((NUM_EXPERTS, D_IN), lambda i: (0, 0)),
            pl.BlockSpec((NUM_EXPERTS, RANK, D_IN), lambda i: (0, 0, 0)),
            pl.BlockSpec((NUM_EXPERTS, D_OUT, RANK), lambda i: (0, 0, 0)),
            pl.BlockSpec((BLK, NUM_EXPERTS), lambda i: (i, 0)),
        ],
        out_specs=[
            pl.BlockSpec((BLK, D_OUT), lambda i: (i, 0)),
            pl.BlockSpec((BLK, NUM_EXPERTS), lambda i: (i, 0)),
        ],
        out_shape=[
            jax.ShapeDtypeStruct((T, D_OUT), jnp.float32),
            jax.ShapeDtypeStruct((T, NUM_EXPERTS), jnp.float32),
        ],
        compiler_params=pltpu.CompilerParams(
            dimension_semantics=("arbitrary",),
        ),
        interpret=interpret,
    )(x, Wg, Wn, W_down.astype(jnp.bfloat16), W_up.astype(jnp.bfloat16),
      noise)
    return out.reshape(b, s, D_OUT), rl


# speedup vs baseline: 1.6029x; 1.6029x over previous
import functools

import jax
import jax.numpy as jnp
from jax.experimental import pallas as pl
from jax.experimental.pallas import tpu as pltpu

NUM_EXPERTS = 8
TOP_K = 2
RANK = 128
D_IN = 2048
D_OUT = 2048
ER = NUM_EXPERTS * RANK
BLK = 512


def _moe_body(x_ref, wg_ref, wn_ref, wd_ref, wu_ref, noise_ref,
              out_ref, rl_ref):
    x = x_ref[...]

    logits = jax.lax.dot_general(
        x, wg_ref[...], (((1,), (1,)), ((), ())),
        preferred_element_type=jnp.float32)
    nlogits = jax.lax.dot_general(
        x, wn_ref[...], (((1,), (1,)), ((), ())),
        preferred_element_type=jnp.float32)
    rl = logits + noise_ref[...] * jax.nn.softplus(nlogits)
    rl_ref[...] = rl

    p = jax.nn.softmax(rl, axis=-1)

    col = jax.lax.broadcasted_iota(jnp.int32, p.shape, 1)
    m1 = jnp.max(p, axis=-1, keepdims=True)
    a1 = jnp.min(jnp.where(p == m1, col, NUM_EXPERTS), axis=-1, keepdims=True)
    first = col == a1
    p_m = jnp.where(first, -jnp.inf, p)
    m2 = jnp.max(p_m, axis=-1, keepdims=True)
    a2 = jnp.min(jnp.where(p_m == m2, col, NUM_EXPERTS), axis=-1, keepdims=True)
    sel = first | (col == a2)
    w = jnp.where(sel, p, 0.0)
    w = w / jnp.sum(w, axis=-1, keepdims=True)

    xb = x.astype(jnp.bfloat16)
    down = jax.lax.dot_general(
        xb, wd_ref[...], (((1,), (1,)), ((), ())),
        preferred_element_type=jnp.float32)
    wexp = jnp.broadcast_to(w[:, :, None],
                            (w.shape[0], NUM_EXPERTS, RANK))
    wexp = wexp.reshape(w.shape[0], ER)
    scaled = (down * wexp).astype(jnp.bfloat16)
    up = jax.lax.dot_general(
        scaled, wu_ref[...], (((1,), (0,)), ((), ())),
        preferred_element_type=jnp.float32)
    out_ref[...] = up


@functools.partial(jax.jit, static_argnames=("interpret",))
def kernel(hidden_states, Wg, Wn, W_down, W_up, interpret=False):
    b, s, d = hidden_states.shape
    T = b * s
    x = hidden_states.reshape(T, d)
    noise = jax.random.normal(jax.random.key(42), (T, NUM_EXPERTS),
                              jnp.float32)
    wd_all = W_down.reshape(ER, D_IN).astype(jnp.bfloat16)
    wu_all = W_up.transpose(0, 2, 1).reshape(ER, D_OUT).astype(jnp.bfloat16)

    grid = (T // BLK,)
    out, rl = pl.pallas_call(
        _moe_body,
        grid=grid,
        in_specs=[
            pl.BlockSpec((BLK, D_IN), lambda i: (i, 0)),
            pl.BlockSpec((NUM_EXPERTS, D_IN), lambda i: (0, 0)),
            pl.BlockSpec((NUM_EXPERTS, D_IN), lambda i: (0, 0)),
            pl.BlockSpec((ER, D_IN), lambda i: (0, 0)),
            pl.BlockSpec((ER, D_OUT), lambda i: (0, 0)),
            pl.BlockSpec((BLK, NUM_EXPERTS), lambda i: (i, 0)),
        ],
        out_specs=[
            pl.BlockSpec((BLK, D_OUT), lambda i: (i, 0)),
            pl.BlockSpec((BLK, NUM_EXPERTS), lambda i: (i, 0)),
        ],
        out_shape=[
            jax.ShapeDtypeStruct((T, D_OUT), jnp.float32),
            jax.ShapeDtypeStruct((T, NUM_EXPERTS), jnp.float32),
        ],
        compiler_params=pltpu.CompilerParams(
            dimension_semantics=("arbitrary",),
        ),
        interpret=interpret,
    )(x, Wg, Wn, wd_all, wu_all, noise)
    return out.reshape(b, s, D_OUT), rl
